# Initial kernel scaffold; baseline (speedup 1.0000x reference)
import jax
import jax.numpy as jnp
from jax.experimental import pallas as pl


def kernel(x, edge_index, pre_W, pre_b, pre_ln_g, pre_ln_b, rel_W, rel_b, root_W, ln_g, ln_b, lin_W, lin_b):
    raise NotImplementedError("write your pallas kernel here")


# speedup vs baseline: 3.2861x; 3.2861x over previous
import functools

import jax
import jax.numpy as jnp
from jax import lax
from jax.experimental import pallas as pl
from jax.experimental.pallas import tpu as pltpu
from jax.experimental.pallas import tpu_sc as plsc

N = 10000
E = 160000
D = 256
H = 128
NC = 2
NS = 16
EPT = E // NS
CHUNK = 128
EPT_PAD = 10240
NCHUNKS = EPT_PAD // CHUNK
ZROWS = 632
NP = NS * ZROWS
OROWS_LAST = N - (NS - 1) * ZROWS

_HIGH = jax.lax.Precision.HIGHEST


def _dot(a, b):
    return jnp.dot(a, b, preferred_element_type=jnp.float32, precision=_HIGH)


def _ln(h, g, b, eps=1e-5):
    mu = jnp.mean(h, axis=-1, keepdims=True)
    var = jnp.mean((h - mu) * (h - mu), axis=-1, keepdims=True)
    return (h - mu) * jax.lax.rsqrt(var + eps) * g + b


def _sc_body(hsplit, src_hbm, dst_hbm, zeros_hbm, agg_hbm,
             src_v, dst_v, rows_v, acc_sh, sem):
    c = lax.axis_index("c")
    s = lax.axis_index("s")
    w = c * NS + s
    pltpu.sync_copy(src_hbm.at[w], src_v)
    pltpu.sync_copy(dst_hbm.at[s], dst_v)
    pltpu.sync_copy(zeros_hbm.at[pl.ds(s * ZROWS, ZROWS)],
                    acc_sh.at[pl.ds(s * ZROWS, ZROWS)])
    plsc.subcore_barrier()

    def step(j, carry):
        pltpu.async_copy(hsplit.at[src_v.at[j]], rows_v, sem).wait()
        pltpu.sync_copy(rows_v, acc_sh.at[dst_v.at[j]], add=True)
        return carry

    lax.fori_loop(0, NCHUNKS, step, 0)
    plsc.subcore_barrier()

    @pl.when(s < NS - 1)
    def _():
        pltpu.sync_copy(acc_sh.at[pl.ds(s * ZROWS, ZROWS)],
                        agg_hbm.at[pl.ds(c * N + s * ZROWS, ZROWS)])

    @pl.when(s == NS - 1)
    def _():
        pltpu.sync_copy(acc_sh.at[pl.ds((NS - 1) * ZROWS, OROWS_LAST)],
                        agg_hbm.at[pl.ds(c * N + (NS - 1) * ZROWS, OROWS_LAST)])


_sc_msg = functools.partial(
    pl.kernel,
    out_type=jax.ShapeDtypeStruct((NC * N, H), jnp.float32),
    mesh=plsc.VectorSubcoreMesh(core_axis_name="c", subcore_axis_name="s",
                                num_cores=NC, num_subcores=NS),
    scratch_types=[
        pltpu.VMEM((NCHUNKS, CHUNK), jnp.int32),
        pltpu.VMEM((NCHUNKS, CHUNK), jnp.int32),
        pltpu.VMEM((CHUNK, H), jnp.float32),
        pltpu.VMEM_SHARED((NP, H), jnp.float32),
        pltpu.SemaphoreType.DMA,
    ],
)(_sc_body)


_BLK = 1000
_GRID = N // _BLK


def _pre_body(x_ref, w_ref, b_ref, g_ref, bt_ref, o_ref):
    h = _dot(x_ref[...], w_ref[...]) + b_ref[...]
    h = _ln(h, g_ref[...], bt_ref[...])
    h = jnp.maximum(h, 0.0)
    o_ref[0] = h[:, :H]
    o_ref[1] = h[:, H:]


def _pre(x, w, b, g, bt):
    return pl.pallas_call(
        _pre_body,
        grid=(_GRID,),
        in_specs=[
            pl.BlockSpec((_BLK, D), lambda i: (i, 0)),
            pl.BlockSpec((D, D), lambda i: (0, 0)),
            pl.BlockSpec((1, D), lambda i: (0, 0)),
            pl.BlockSpec((1, D), lambda i: (0, 0)),
            pl.BlockSpec((1, D), lambda i: (0, 0)),
        ],
        out_specs=pl.BlockSpec((NC, _BLK, H), lambda i: (0, i, 0)),
        out_shape=jax.ShapeDtypeStruct((NC, N, H), jnp.float32),
    )(x, w, b, g, bt)


def _layer_body(agg_ref, hs_ref, rw_ref, rootw_ref, rb_ref, g_ref, bt_ref,
                o_ref):
    h = jnp.concatenate([hs_ref[0], hs_ref[1]], axis=1)
    rw = rw_ref[...]
    z = (_dot(agg_ref[0], rw[:H]) + _dot(agg_ref[1], rw[H:])
         + _dot(h, rootw_ref[...]) + rb_ref[...])
    z = jnp.maximum(z, 0.0)
    z = _ln(z, g_ref[...], bt_ref[...])
    o_ref[0] = z[:, :H]
    o_ref[1] = z[:, H:]


def _layer(agg, hs, rw, rootw, rb, g, bt):
    return pl.pallas_call(
        _layer_body,
        grid=(_GRID,),
        in_specs=[
            pl.BlockSpec((NC, _BLK, H), lambda i: (0, i, 0)),
            pl.BlockSpec((NC, _BLK, H), lambda i: (0, i, 0)),
            pl.BlockSpec((D, D), lambda i: (0, 0)),
            pl.BlockSpec((D, D), lambda i: (0, 0)),
            pl.BlockSpec((1, D), lambda i: (0, 0)),
            pl.BlockSpec((1, D), lambda i: (0, 0)),
            pl.BlockSpec((1, D), lambda i: (0, 0)),
        ],
        out_specs=pl.BlockSpec((NC, _BLK, H), lambda i: (0, i, 0)),
        out_shape=jax.ShapeDtypeStruct((NC, N, H), jnp.float32),
    )(agg, hs, rw, rootw, rb, g, bt)


def _final_body(x_ref, hs_ref, lw_ref, lb_ref, o_ref):
    h = jnp.concatenate([hs_ref[0], hs_ref[1]], axis=1)
    y = _dot(h, lw_ref[...]) + lb_ref[...]
    o_ref[...] = jnp.concatenate([x_ref[...], y], axis=1)


def _final(x, hs, lw, lb):
    d_out = D + lw.shape[1]
    return pl.pallas_call(
        _final_body,
        grid=(_GRID,),
        in_specs=[
            pl.BlockSpec((_BLK, D), lambda i: (i, 0)),
            pl.BlockSpec((NC, _BLK, H), lambda i: (0, i, 0)),
            pl.BlockSpec((D, lw.shape[1]), lambda i: (0, 0)),
            pl.BlockSpec((1, lw.shape[1]), lambda i: (0, 0)),
        ],
        out_specs=pl.BlockSpec((_BLK, d_out), lambda i: (i, 0)),
        out_shape=jax.ShapeDtypeStruct((N, d_out), jnp.float32),
    )(x, hs, lw, lb)


def kernel(x, edge_index, pre_W, pre_b, pre_ln_g, pre_ln_b,
           rel_W, rel_b, root_W, ln_g, ln_b, lin_W, lin_b):
    src = edge_index[0]
    dst = edge_index[1]
    srcp = jnp.pad(src.reshape(NS, EPT), ((0, 0), (0, EPT_PAD - EPT)))
    dstp = jnp.pad(dst.reshape(NS, EPT), ((0, 0), (0, EPT_PAD - EPT)),
                   constant_values=N)
    src_ix = jnp.stack([srcp, srcp + N]).astype(jnp.int32)
    src_ix = src_ix.reshape(NC * NS, NCHUNKS, CHUNK)
    dst_ix = dstp.astype(jnp.int32).reshape(NS, NCHUNKS, CHUNK)
    zeros = jnp.zeros((NP, H), jnp.float32)

    hs = _pre(x, pre_W, pre_b.reshape(1, D), pre_ln_g.reshape(1, D),
              pre_ln_b.reshape(1, D))
    for i in range(rel_W.shape[0]):
        agg = _sc_msg(hs.reshape(NC * N, H), src_ix, dst_ix, zeros)
        hs = _layer(agg.reshape(NC, N, H), hs, rel_W[i], root_W[i],
                    rel_b[i].reshape(1, D), ln_g[i].reshape(1, D),
                    ln_b[i].reshape(1, D))
    return _final(x, hs, lin_W, lin_b.reshape(1, lin_W.shape[1]))


# speedup vs baseline: 3.3013x; 1.0046x over previous
import functools

import jax
import jax.numpy as jnp
from jax import lax
from jax.experimental import pallas as pl
from jax.experimental.pallas import tpu as pltpu
from jax.experimental.pallas import tpu_sc as plsc

N = 10000
E = 160000
D = 256
H = 128
NC = 2
NS = 16
EPT = E // NS
CHUNK = 64
EPT_PAD = 10240
NROWS = EPT_PAD // (2 * CHUNK)
ZROWS = 632
NP = NS * ZROWS
OROWS_LAST = N - (NS - 1) * ZROWS

_HIGH = jax.lax.Precision.HIGHEST


def _dot(a, b):
    return jnp.dot(a, b, preferred_element_type=jnp.float32, precision=_HIGH)


def _ln(h, g, b, eps=1e-5):
    mu = jnp.mean(h, axis=-1, keepdims=True)
    var = jnp.mean((h - mu) * (h - mu), axis=-1, keepdims=True)
    return (h - mu) * jax.lax.rsqrt(var + eps) * g + b


def _sc_body(hsplit, src_hbm, dst_hbm, zeros_hbm, agg_hbm,
             src_v, dst_v, rows0_v, rows1_v, acc_sh, sem0, sem1):
    c = lax.axis_index("c")
    s = lax.axis_index("s")
    w = c * NS + s
    pltpu.sync_copy(src_hbm.at[w], src_v)
    pltpu.sync_copy(dst_hbm.at[s], dst_v)
    pltpu.sync_copy(zeros_hbm.at[pl.ds(s * ZROWS, ZROWS)],
                    acc_sh.at[pl.ds(s * ZROWS, ZROWS)])
    plsc.subcore_barrier()

    pltpu.async_copy(hsplit.at[src_v.at[0, pl.ds(0, CHUNK)]], rows0_v, sem0)

    def step(r, carry):
        pltpu.make_async_copy(hsplit.at[src_v.at[r, pl.ds(0, CHUNK)]],
                              rows0_v, sem0).wait()
        pltpu.async_copy(hsplit.at[src_v.at[r, pl.ds(CHUNK, CHUNK)]],
                         rows1_v, sem1)
        pltpu.sync_copy(rows0_v, acc_sh.at[dst_v.at[r, pl.ds(0, CHUNK)]],
                        add=True)
        pltpu.make_async_copy(hsplit.at[src_v.at[r, pl.ds(CHUNK, CHUNK)]],
                              rows1_v, sem1).wait()

        @pl.when(r + 1 < NROWS)
        def _():
            pltpu.async_copy(hsplit.at[src_v.at[r + 1, pl.ds(0, CHUNK)]],
                             rows0_v, sem0)

        pltpu.sync_copy(rows1_v, acc_sh.at[dst_v.at[r, pl.ds(CHUNK, CHUNK)]],
                        add=True)
        return carry

    lax.fori_loop(0, NROWS, step, 0)
    plsc.subcore_barrier()

    @pl.when(s < NS - 1)
    def _():
        pltpu.sync_copy(acc_sh.at[pl.ds(s * ZROWS, ZROWS)],
                        agg_hbm.at[pl.ds(c * N + s * ZROWS, ZROWS)])

    @pl.when(s == NS - 1)
    def _():
        pltpu.sync_copy(acc_sh.at[pl.ds((NS - 1) * ZROWS, OROWS_LAST)],
                        agg_hbm.at[pl.ds(c * N + (NS - 1) * ZROWS, OROWS_LAST)])


_sc_msg = functools.partial(
    pl.kernel,
    out_type=jax.ShapeDtypeStruct((NC * N, H), jnp.float32),
    mesh=plsc.VectorSubcoreMesh(core_axis_name="c", subcore_axis_name="s",
                                num_cores=NC, num_subcores=NS),
    scratch_types=[
        pltpu.VMEM((NROWS, 2 * CHUNK), jnp.int32),
        pltpu.VMEM((NROWS, 2 * CHUNK), jnp.int32),
        pltpu.VMEM((CHUNK, H), jnp.float32),
        pltpu.VMEM((CHUNK, H), jnp.float32),
        pltpu.VMEM_SHARED((NP, H), jnp.float32),
        pltpu.SemaphoreType.DMA,
        pltpu.SemaphoreType.DMA,
    ],
)(_sc_body)


_BLK = 1000
_GRID = N // _BLK


def _pre_body(x_ref, w_ref, b_ref, g_ref, bt_ref, o_ref):
    h = _dot(x_ref[...], w_ref[...]) + b_ref[...]
    h = _ln(h, g_ref[...], bt_ref[...])
    h = jnp.maximum(h, 0.0)
    o_ref[0] = h[:, :H]
    o_ref[1] = h[:, H:]


def _pre(x, w, b, g, bt):
    return pl.pallas_call(
        _pre_body,
        grid=(_GRID,),
        in_specs=[
            pl.BlockSpec((_BLK, D), lambda i: (i, 0)),
            pl.BlockSpec((D, D), lambda i: (0, 0)),
            pl.BlockSpec((1, D), lambda i: (0, 0)),
            pl.BlockSpec((1, D), lambda i: (0, 0)),
            pl.BlockSpec((1, D), lambda i: (0, 0)),
        ],
        out_specs=pl.BlockSpec((NC, _BLK, H), lambda i: (0, i, 0)),
        out_shape=jax.ShapeDtypeStruct((NC, N, H), jnp.float32),
    )(x, w, b, g, bt)


def _layer_body(agg_ref, hs_ref, rw_ref, rootw_ref, rb_ref, g_ref, bt_ref,
                o_ref):
    h = jnp.concatenate([hs_ref[0], hs_ref[1]], axis=1)
    rw = rw_ref[...]
    z = (_dot(agg_ref[0], rw[:H]) + _dot(agg_ref[1], rw[H:])
         + _dot(h, rootw_ref[...]) + rb_ref[...])
    z = jnp.maximum(z, 0.0)
    z = _ln(z, g_ref[...], bt_ref[...])
    o_ref[0] = z[:, :H]
    o_ref[1] = z[:, H:]


def _layer(agg, hs, rw, rootw, rb, g, bt):
    return pl.pallas_call(
        _layer_body,
        grid=(_GRID,),
        in_specs=[
            pl.BlockSpec((NC, _BLK, H), lambda i: (0, i, 0)),
            pl.BlockSpec((NC, _BLK, H), lambda i: (0, i, 0)),
            pl.BlockSpec((D, D), lambda i: (0, 0)),
            pl.BlockSpec((D, D), lambda i: (0, 0)),
            pl.BlockSpec((1, D), lambda i: (0, 0)),
            pl.BlockSpec((1, D), lambda i: (0, 0)),
            pl.BlockSpec((1, D), lambda i: (0, 0)),
        ],
        out_specs=pl.BlockSpec((NC, _BLK, H), lambda i: (0, i, 0)),
        out_shape=jax.ShapeDtypeStruct((NC, N, H), jnp.float32),
    )(agg, hs, rw, rootw, rb, g, bt)


def _final_body(x_ref, hs_ref, lw_ref, lb_ref, o_ref):
    h = jnp.concatenate([hs_ref[0], hs_ref[1]], axis=1)
    y = _dot(h, lw_ref[...]) + lb_ref[...]
    o_ref[...] = jnp.concatenate([x_ref[...], y], axis=1)


def _final(x, hs, lw, lb):
    d_out = D + lw.shape[1]
    return pl.pallas_call(
        _final_body,
        grid=(_GRID,),
        in_specs=[
            pl.BlockSpec((_BLK, D), lambda i: (i, 0)),
            pl.BlockSpec((NC, _BLK, H), lambda i: (0, i, 0)),
            pl.BlockSpec((D, lw.shape[1]), lambda i: (0, 0)),
            pl.BlockSpec((1, lw.shape[1]), lambda i: (0, 0)),
        ],
        out_specs=pl.BlockSpec((_BLK, d_out), lambda i: (i, 0)),
        out_shape=jax.ShapeDtypeStruct((N, d_out), jnp.float32),
    )(x, hs, lw, lb)


def kernel(x, edge_index, pre_W, pre_b, pre_ln_g, pre_ln_b,
           rel_W, rel_b, root_W, ln_g, ln_b, lin_W, lin_b):
    src = edge_index[0]
    dst = edge_index[1]
    srcp = jnp.pad(src.reshape(NS, EPT), ((0, 0), (0, EPT_PAD - EPT)))
    dstp = jnp.pad(dst.reshape(NS, EPT), ((0, 0), (0, EPT_PAD - EPT)),
                   constant_values=N)
    src_ix = jnp.stack([srcp, srcp + N]).astype(jnp.int32)
    src_ix = src_ix.reshape(NC * NS, NROWS, 2 * CHUNK)
    dst_ix = dstp.astype(jnp.int32).reshape(NS, NROWS, 2 * CHUNK)
    zeros = jnp.zeros((NP, H), jnp.float32)

    hs = _pre(x, pre_W, pre_b.reshape(1, D), pre_ln_g.reshape(1, D),
              pre_ln_b.reshape(1, D))
    for i in range(rel_W.shape[0]):
        agg = _sc_msg(hs.reshape(NC * N, H), src_ix, dst_ix, zeros)
        hs = _layer(agg.reshape(NC, N, H), hs, rel_W[i], root_W[i],
                    rel_b[i].reshape(1, D), ln_g[i].reshape(1, D),
                    ln_b[i].reshape(1, D))
    return _final(x, hs, lin_W, lin_b.reshape(1, lin_W.shape[1]))


# speedup vs baseline: 4.0229x; 1.2186x over previous
import functools

import jax
import jax.numpy as jnp
from jax import lax
from jax.experimental import pallas as pl
from jax.experimental.pallas import tpu as pltpu
from jax.experimental.pallas import tpu_sc as plsc

N = 10000
E = 160000
D = 256
H = 128
NC = 2
NS = 16
EPT = E // NS
CHUNK = 64
EPT_PAD = 10240
NROWS = EPT_PAD // (2 * CHUNK)
ZROWS = 632
NP = NS * ZROWS
OROWS_LAST = N - (NS - 1) * ZROWS

_HIGH = jax.lax.Precision.HIGHEST


def _dot(a, b):
    return jnp.dot(a, b, preferred_element_type=jnp.float32, precision=_HIGH)


def _ln(h, g, b, eps=1e-5):
    mu = jnp.mean(h, axis=-1, keepdims=True)
    var = jnp.mean((h - mu) * (h - mu), axis=-1, keepdims=True)
    return (h - mu) * jax.lax.rsqrt(var + eps) * g + b


def _unpack(packed_v, row, col, out_ref, out_row, shift, base):
    for q in range(CHUNK // 16):
        v = packed_v[row, pl.ds(col + 16 * q, 16)]
        f = lax.shift_right_logical(v, shift) & 0xFFFF
        out_ref[out_row, pl.ds(16 * q, 16)] = f + base


def _sc_body(hsplit, packed_hbm, zeros_hbm, agg_hbm,
             packed_v, sstage_v, dstage_v,
             rows0_v, rows1_v, rows2_v, rows3_v, acc_sh,
             sem0, sem1, sem2, sem3):
    c = lax.axis_index("c")
    s = lax.axis_index("s")
    coff = c * N
    pltpu.sync_copy(packed_hbm.at[s], packed_v)
    pltpu.sync_copy(zeros_hbm.at[pl.ds(s * ZROWS, ZROWS)],
                    acc_sh.at[pl.ds(s * ZROWS, ZROWS)])
    plsc.subcore_barrier()

    bufs = [rows0_v, rows1_v, rows2_v, rows3_v]
    sems = [sem0, sem1, sem2, sem3]

    for i in range(4):
        _unpack(packed_v, i >> 1, (i & 1) * CHUNK, sstage_v, i, 0, coff)
        pltpu.async_copy(hsplit.at[sstage_v.at[i, pl.ds(0, CHUNK)]],
                         bufs[i], sems[i])

    def step(t, carry):
        r0 = 2 * t
        for i in range(4):
            row = r0 + (i >> 1)
            col = (i & 1) * CHUNK
            pltpu.make_async_copy(hsplit.at[sstage_v.at[i, pl.ds(0, CHUNK)]],
                                  bufs[i], sems[i]).wait()
            _unpack(packed_v, row, col, dstage_v, 0, 16, 0)
            pltpu.sync_copy(bufs[i], acc_sh.at[dstage_v.at[0, pl.ds(0, CHUNK)]],
                            add=True)

            @pl.when(t + 1 < NROWS // 2)
            def _():
                _unpack(packed_v, row + 2, col, sstage_v, i, 0, coff)
                pltpu.async_copy(hsplit.at[sstage_v.at[i, pl.ds(0, CHUNK)]],
                                 bufs[i], sems[i])

        return carry

    lax.fori_loop(0, NROWS // 2, step, 0)
    plsc.subcore_barrier()

    @pl.when(s < NS - 1)
    def _():
        pltpu.sync_copy(acc_sh.at[pl.ds(s * ZROWS, ZROWS)],
                        agg_hbm.at[pl.ds(c * N + s * ZROWS, ZROWS)])

    @pl.when(s == NS - 1)
    def _():
        pltpu.sync_copy(acc_sh.at[pl.ds((NS - 1) * ZROWS, OROWS_LAST)],
                        agg_hbm.at[pl.ds(c * N + (NS - 1) * ZROWS, OROWS_LAST)])


_sc_msg = functools.partial(
    pl.kernel,
    out_type=jax.ShapeDtypeStruct((NC * N, H), jnp.float32),
    mesh=plsc.VectorSubcoreMesh(core_axis_name="c", subcore_axis_name="s",
                                num_cores=NC, num_subcores=NS),
    scratch_types=[
        pltpu.VMEM((NROWS, 2 * CHUNK), jnp.int32),
        pltpu.VMEM((4, 2 * CHUNK), jnp.int32),
        pltpu.VMEM((1, 2 * CHUNK), jnp.int32),
        pltpu.VMEM((CHUNK, H), jnp.float32),
        pltpu.VMEM((CHUNK, H), jnp.float32),
        pltpu.VMEM((CHUNK, H), jnp.float32),
        pltpu.VMEM((CHUNK, H), jnp.float32),
        pltpu.VMEM_SHARED((NP, H), jnp.float32),
        pltpu.SemaphoreType.DMA,
        pltpu.SemaphoreType.DMA,
        pltpu.SemaphoreType.DMA,
        pltpu.SemaphoreType.DMA,
    ],
)(_sc_body)


_BLK = 1000
_GRID = N // _BLK


def _pre_body(x_ref, w_ref, b_ref, g_ref, bt_ref, o_ref):
    h = _dot(x_ref[...], w_ref[...]) + b_ref[...]
    h = _ln(h, g_ref[...], bt_ref[...])
    h = jnp.maximum(h, 0.0)
    o_ref[0] = h[:, :H]
    o_ref[1] = h[:, H:]


def _pre(x, w, b, g, bt):
    return pl.pallas_call(
        _pre_body,
        grid=(_GRID,),
        in_specs=[
            pl.BlockSpec((_BLK, D), lambda i: (i, 0)),
            pl.BlockSpec((D, D), lambda i: (0, 0)),
            pl.BlockSpec((1, D), lambda i: (0, 0)),
            pl.BlockSpec((1, D), lambda i: (0, 0)),
            pl.BlockSpec((1, D), lambda i: (0, 0)),
        ],
        out_specs=pl.BlockSpec((NC, _BLK, H), lambda i: (0, i, 0)),
        out_shape=jax.ShapeDtypeStruct((NC, N, H), jnp.float32),
    )(x, w, b, g, bt)


def _layer_body(agg_ref, hs_ref, rw_ref, rootw_ref, rb_ref, g_ref, bt_ref,
                o_ref):
    h = jnp.concatenate([hs_ref[0], hs_ref[1]], axis=1)
    rw = rw_ref[...]
    z = (_dot(agg_ref[0], rw[:H]) + _dot(agg_ref[1], rw[H:])
         + _dot(h, rootw_ref[...]) + rb_ref[...])
    z = jnp.maximum(z, 0.0)
    z = _ln(z, g_ref[...], bt_ref[...])
    o_ref[0] = z[:, :H]
    o_ref[1] = z[:, H:]


def _layer(agg, hs, rw, rootw, rb, g, bt):
    return pl.pallas_call(
        _layer_body,
        grid=(_GRID,),
        in_specs=[
            pl.BlockSpec((NC, _BLK, H), lambda i: (0, i, 0)),
            pl.BlockSpec((NC, _BLK, H), lambda i: (0, i, 0)),
            pl.BlockSpec((D, D), lambda i: (0, 0)),
            pl.BlockSpec((D, D), lambda i: (0, 0)),
            pl.BlockSpec((1, D), lambda i: (0, 0)),
            pl.BlockSpec((1, D), lambda i: (0, 0)),
            pl.BlockSpec((1, D), lambda i: (0, 0)),
        ],
        out_specs=pl.BlockSpec((NC, _BLK, H), lambda i: (0, i, 0)),
        out_shape=jax.ShapeDtypeStruct((NC, N, H), jnp.float32),
    )(agg, hs, rw, rootw, rb, g, bt)


def _final_body(x_ref, hs_ref, lw_ref, lb_ref, o_ref):
    h = jnp.concatenate([hs_ref[0], hs_ref[1]], axis=1)
    y = _dot(h, lw_ref[...]) + lb_ref[...]
    o_ref[...] = jnp.concatenate([x_ref[...], y], axis=1)


def _final(x, hs, lw, lb):
    d_out = D + lw.shape[1]
    return pl.pallas_call(
        _final_body,
        grid=(_GRID,),
        in_specs=[
            pl.BlockSpec((_BLK, D), lambda i: (i, 0)),
            pl.BlockSpec((NC, _BLK, H), lambda i: (0, i, 0)),
            pl.BlockSpec((D, lw.shape[1]), lambda i: (0, 0)),
            pl.BlockSpec((1, lw.shape[1]), lambda i: (0, 0)),
        ],
        out_specs=pl.BlockSpec((_BLK, d_out), lambda i: (i, 0)),
        out_shape=jax.ShapeDtypeStruct((N, d_out), jnp.float32),
    )(x, hs, lw, lb)


def kernel(x, edge_index, pre_W, pre_b, pre_ln_g, pre_ln_b,
           rel_W, rel_b, root_W, ln_g, ln_b, lin_W, lin_b):
    src = edge_index[0]
    dst = edge_index[1]
    srcp = jnp.pad(src.reshape(NS, EPT), ((0, 0), (0, EPT_PAD - EPT)))
    dstp = jnp.pad(dst.reshape(NS, EPT), ((0, 0), (0, EPT_PAD - EPT)),
                   constant_values=N)
    packed = (srcp.astype(jnp.int32) | (dstp.astype(jnp.int32) << 16))
    packed = packed.reshape(NS, NROWS, 2 * CHUNK)
    zeros = jnp.zeros((NP, H), jnp.float32)

    hs = _pre(x, pre_W, pre_b.reshape(1, D), pre_ln_g.reshape(1, D),
              pre_ln_b.reshape(1, D))
    for i in range(rel_W.shape[0]):
        agg = _sc_msg(hs.reshape(NC * N, H), packed, zeros)
        hs = _layer(agg.reshape(NC, N, H), hs, rel_W[i], root_W[i],
                    rel_b[i].reshape(1, D), ln_g[i].reshape(1, D),
                    ln_b[i].reshape(1, D))
    return _final(x, hs, lin_W, lin_b.reshape(1, lin_W.shape[1]))


# speedup vs baseline: 4.0529x; 1.0075x over previous
import functools

import jax
import jax.numpy as jnp
from jax import lax
from jax.experimental import pallas as pl
from jax.experimental.pallas import tpu as pltpu
from jax.experimental.pallas import tpu_sc as plsc

N = 10000
E = 160000
D = 256
H = 128
NC = 2
NS = 16
EPT = E // NS
CHUNK = 64
EPT_PAD = 10240
NROWS = EPT_PAD // (2 * CHUNK)
ZROWS = 632
NP = NS * ZROWS
OROWS_LAST = N - (NS - 1) * ZROWS

_HIGH = jax.lax.Precision.HIGHEST


def _dot(a, b):
    return jnp.dot(a, b, preferred_element_type=jnp.float32, precision=_HIGH)


def _ln(h, g, b, eps=1e-5):
    mu = jnp.mean(h, axis=-1, keepdims=True)
    var = jnp.mean((h - mu) * (h - mu), axis=-1, keepdims=True)
    return (h - mu) * jax.lax.rsqrt(var + eps) * g + b


def _unpack(packed_v, row, col, out_ref, out_row, shift, base):
    for q in range(CHUNK // 16):
        v = packed_v[row, pl.ds(col + 16 * q, 16)]
        f = lax.shift_right_logical(v, shift) & 0xFFFF
        out_ref[out_row, pl.ds(16 * q, 16)] = f + base


def _sc_body(hsplit, packed_hbm, zeros_hbm, agg_hbm,
             packed_v, sstage_v, dstage_v,
             rows0_v, rows1_v, rows2_v, rows3_v, acc_sh,
             sem0, sem1, sem2, sem3):
    c = lax.axis_index("c")
    s = lax.axis_index("s")
    coff = c * N
    pltpu.sync_copy(packed_hbm.at[s], packed_v)
    pltpu.sync_copy(zeros_hbm.at[pl.ds(s * ZROWS, ZROWS)],
                    acc_sh.at[pl.ds(s * ZROWS, ZROWS)])
    plsc.subcore_barrier()

    bufs = [rows0_v, rows1_v, rows2_v, rows3_v]
    sems = [sem0, sem1, sem2, sem3]

    for i in range(4):
        _unpack(packed_v, i >> 1, (i & 1) * CHUNK, sstage_v, i, 0, coff)
        pltpu.async_copy(hsplit.at[sstage_v.at[i, pl.ds(0, CHUNK)]],
                         bufs[i], sems[i])

    def step(t, carry):
        r0 = 2 * t
        for i in range(4):
            row = r0 + (i >> 1)
            col = (i & 1) * CHUNK
            pltpu.make_async_copy(hsplit.at[sstage_v.at[i, pl.ds(0, CHUNK)]],
                                  bufs[i], sems[i]).wait()
            _unpack(packed_v, row, col, dstage_v, 0, 16, 0)
            pltpu.sync_copy(bufs[i], acc_sh.at[dstage_v.at[0, pl.ds(0, CHUNK)]],
                            add=True)

            @pl.when(t + 1 < NROWS // 2)
            def _():
                _unpack(packed_v, row + 2, col, sstage_v, i, 0, coff)
                pltpu.async_copy(hsplit.at[sstage_v.at[i, pl.ds(0, CHUNK)]],
                                 bufs[i], sems[i])

        return carry

    lax.fori_loop(0, NROWS // 2, step, 0)
    plsc.subcore_barrier()

    @pl.when(s < NS - 1)
    def _():
        pltpu.sync_copy(acc_sh.at[pl.ds(s * ZROWS, ZROWS)],
                        agg_hbm.at[pl.ds(c * N + s * ZROWS, ZROWS)])

    @pl.when(s == NS - 1)
    def _():
        pltpu.sync_copy(acc_sh.at[pl.ds((NS - 1) * ZROWS, OROWS_LAST)],
                        agg_hbm.at[pl.ds(c * N + (NS - 1) * ZROWS, OROWS_LAST)])


_sc_msg = functools.partial(
    pl.kernel,
    out_type=jax.ShapeDtypeStruct((NC * N, H), jnp.float32),
    mesh=plsc.VectorSubcoreMesh(core_axis_name="c", subcore_axis_name="s",
                                num_cores=NC, num_subcores=NS),
    scratch_types=[
        pltpu.VMEM((NROWS, 2 * CHUNK), jnp.int32),
        pltpu.VMEM((4, 2 * CHUNK), jnp.int32),
        pltpu.VMEM((1, 2 * CHUNK), jnp.int32),
        pltpu.VMEM((CHUNK, H), jnp.float32),
        pltpu.VMEM((CHUNK, H), jnp.float32),
        pltpu.VMEM((CHUNK, H), jnp.float32),
        pltpu.VMEM((CHUNK, H), jnp.float32),
        pltpu.VMEM_SHARED((NP, H), jnp.float32),
        pltpu.SemaphoreType.DMA,
        pltpu.SemaphoreType.DMA,
        pltpu.SemaphoreType.DMA,
        pltpu.SemaphoreType.DMA,
    ],
)(_sc_body)


_BLK = 1000
_GRID = N // _BLK


def _pre_body(x_ref, w_ref, b_ref, g_ref, bt_ref, o_ref):
    h = _dot(x_ref[...], w_ref[...]) + b_ref[...]
    h = _ln(h, g_ref[...], bt_ref[...])
    h = jnp.maximum(h, 0.0)
    o_ref[0] = h[:, :H]
    o_ref[1] = h[:, H:]


def _pre(x, w, b, g, bt):
    return pl.pallas_call(
        _pre_body,
        grid=(_GRID,),
        in_specs=[
            pl.BlockSpec((_BLK, D), lambda i: (i, 0)),
            pl.BlockSpec((D, D), lambda i: (0, 0)),
            pl.BlockSpec((1, D), lambda i: (0, 0)),
            pl.BlockSpec((1, D), lambda i: (0, 0)),
            pl.BlockSpec((1, D), lambda i: (0, 0)),
        ],
        out_specs=pl.BlockSpec((NC, _BLK, H), lambda i: (0, i, 0)),
        out_shape=jax.ShapeDtypeStruct((NC, N, H), jnp.float32),
    )(x, w, b, g, bt)


def _root_body(hs_ref, rootw_ref, rb_ref, o_ref):
    h = jnp.concatenate([hs_ref[0], hs_ref[1]], axis=1)
    z = _dot(h, rootw_ref[...]) + rb_ref[...]
    o_ref[0] = z[:, :H]
    o_ref[1] = z[:, H:]


def _root(hs, rootw, rb):
    return pl.pallas_call(
        _root_body,
        grid=(_GRID,),
        in_specs=[
            pl.BlockSpec((NC, _BLK, H), lambda i: (0, i, 0)),
            pl.BlockSpec((D, D), lambda i: (0, 0)),
            pl.BlockSpec((1, D), lambda i: (0, 0)),
        ],
        out_specs=pl.BlockSpec((NC, _BLK, H), lambda i: (0, i, 0)),
        out_shape=jax.ShapeDtypeStruct((NC, N, H), jnp.float32),
    )(hs, rootw, rb)


def _layer_body(agg_ref, rt_ref, rw_ref, g_ref, bt_ref, o_ref):
    rt = jnp.concatenate([rt_ref[0], rt_ref[1]], axis=1)
    rw = rw_ref[...]
    z = _dot(agg_ref[0], rw[:H]) + _dot(agg_ref[1], rw[H:]) + rt
    z = jnp.maximum(z, 0.0)
    z = _ln(z, g_ref[...], bt_ref[...])
    o_ref[0] = z[:, :H]
    o_ref[1] = z[:, H:]


def _layer(agg, rt, rw, g, bt):
    return pl.pallas_call(
        _layer_body,
        grid=(_GRID,),
        in_specs=[
            pl.BlockSpec((NC, _BLK, H), lambda i: (0, i, 0)),
            pl.BlockSpec((NC, _BLK, H), lambda i: (0, i, 0)),
            pl.BlockSpec((D, D), lambda i: (0, 0)),
            pl.BlockSpec((1, D), lambda i: (0, 0)),
            pl.BlockSpec((1, D), lambda i: (0, 0)),
        ],
        out_specs=pl.BlockSpec((NC, _BLK, H), lambda i: (0, i, 0)),
        out_shape=jax.ShapeDtypeStruct((NC, N, H), jnp.float32),
    )(agg, rt, rw, g, bt)


def _final_body(x_ref, hs_ref, lw_ref, lb_ref, o_ref):
    h = jnp.concatenate([hs_ref[0], hs_ref[1]], axis=1)
    y = _dot(h, lw_ref[...]) + lb_ref[...]
    o_ref[...] = jnp.concatenate([x_ref[...], y], axis=1)


def _final(x, hs, lw, lb):
    d_out = D + lw.shape[1]
    return pl.pallas_call(
        _final_body,
        grid=(_GRID,),
        in_specs=[
            pl.BlockSpec((_BLK, D), lambda i: (i, 0)),
            pl.BlockSpec((NC, _BLK, H), lambda i: (0, i, 0)),
            pl.BlockSpec((D, lw.shape[1]), lambda i: (0, 0)),
            pl.BlockSpec((1, lw.shape[1]), lambda i: (0, 0)),
        ],
        out_specs=pl.BlockSpec((_BLK, d_out), lambda i: (i, 0)),
        out_shape=jax.ShapeDtypeStruct((N, d_out), jnp.float32),
    )(x, hs, lw, lb)


def kernel(x, edge_index, pre_W, pre_b, pre_ln_g, pre_ln_b,
           rel_W, rel_b, root_W, ln_g, ln_b, lin_W, lin_b):
    src = edge_index[0]
    dst = edge_index[1]
    srcp = jnp.pad(src.reshape(NS, EPT), ((0, 0), (0, EPT_PAD - EPT)))
    dstp = jnp.pad(dst.reshape(NS, EPT), ((0, 0), (0, EPT_PAD - EPT)),
                   constant_values=N)
    packed = (srcp.astype(jnp.int32) | (dstp.astype(jnp.int32) << 16))
    packed = packed.reshape(NS, NROWS, 2 * CHUNK)
    zeros = jnp.zeros((NP, H), jnp.float32)

    hs = _pre(x, pre_W, pre_b.reshape(1, D), pre_ln_g.reshape(1, D),
              pre_ln_b.reshape(1, D))
    for i in range(rel_W.shape[0]):
        rooty = _root(hs, root_W[i], rel_b[i].reshape(1, D))
        agg = _sc_msg(hs.reshape(NC * N, H), packed, zeros)
        hs = _layer(agg.reshape(NC, N, H), rooty, rel_W[i],
                    ln_g[i].reshape(1, D), ln_b[i].reshape(1, D))
    return _final(x, hs, lin_W, lin_b.reshape(1, lin_W.shape[1]))
